# baseline (speedup 1.0000x reference)
import functools

import jax
import jax.numpy as jnp
from jax import lax
from jax.experimental import pallas as pl
from jax.experimental.pallas import tpu as pltpu
from jax.experimental.pallas import tpu_sc as plsc

NC = 2
NS = 16
LANES = 16
EB = 128
GC = 8
BR = 1000


def _sc_mesh():
  return plsc.VectorSubcoreMesh(
      core_axis_name="c", subcore_axis_name="s", num_cores=NC,
      num_subcores=NS)


def _make_deg_kernel(N, CH):
  @functools.partial(
      pl.kernel,
      out_type=jax.ShapeDtypeStruct((NC, N), jnp.float32),
      mesh=_sc_mesh(),
      scratch_types=[
          pltpu.VMEM((CH, EB), jnp.int32),
          pltpu.VMEM((CH, EB), jnp.float32),
          pltpu.VMEM_SHARED((N,), jnp.float32),
      ],
  )
  def deg_kernel(dst_hbm, w_hbm, zeros_hbm, out_hbm, dstv, wv, deg_sh):
    cid = lax.axis_index("c")
    sid = lax.axis_index("s")
    wid = sid * NC + cid
    pltpu.sync_copy(dst_hbm.at[wid], dstv)
    pltpu.sync_copy(w_hbm.at[wid], wv)

    @pl.when(sid == 0)
    def _():
      pltpu.sync_copy(zeros_hbm, deg_sh)

    plsc.subcore_barrier()

    def chunk(c, carry):
      pltpu.sync_copy(wv.at[c], deg_sh.at[dstv.at[c]], add=True)
      return carry

    lax.fori_loop(0, CH, chunk, 0)
    plsc.subcore_barrier()

    @pl.when(sid == 0)
    def _():
      pltpu.sync_copy(deg_sh, out_hbm.at[cid])

  return deg_kernel


def _make_agg_kernel(NP, CH, D):
  rpt = NP // NS

  assert CH % (2 * GC) == 0
  NG = CH // GC

  @functools.partial(
      pl.kernel,
      out_type=jax.ShapeDtypeStruct((NC, NP, D), jnp.float32),
      mesh=_sc_mesh(),
      scratch_types=[
          pltpu.VMEM((2, GC, EB), jnp.int32),
          pltpu.VMEM((2, GC, EB), jnp.int32),
          pltpu.VMEM((CH, EB), jnp.float32),
          pltpu.VMEM((EB, D), jnp.float32),
          pltpu.VMEM((EB, D), jnp.float32),
          pltpu.VMEM_SHARED((NP, D), jnp.float32),
          pltpu.SemaphoreType.DMA,
          pltpu.SemaphoreType.DMA,
          pltpu.SemaphoreType.DMA,
          pltpu.SemaphoreType.DMA,
          pltpu.SemaphoreType.DMA,
      ],
  )
  def agg_kernel(xw_hbm, src_hbm, dst_hbm, w_hbm, zrows_hbm, out_hbm,
                 sidx, didx, wv, rows0, rows1, acc,
                 gsem0, gsem1, isem, ssem0, ssem1):
    cid = lax.axis_index("c")
    sid = lax.axis_index("s")
    wid = sid * NC + cid

    pltpu.sync_copy(w_hbm.at[wid], wv)
    pltpu.sync_copy(src_hbm.at[wid].at[0], sidx.at[0])
    pltpu.sync_copy(dst_hbm.at[wid].at[0], didx.at[0])
    pltpu.async_copy(xw_hbm.at[sidx.at[0].at[0]], rows0, gsem0)
    pltpu.sync_copy(zrows_hbm, acc.at[pl.ds(sid * rpt, rpt)])
    plsc.subcore_barrier()

    def scale(rbuf, c):
      def grp16(g, gcarry):
        wg = wv[c, pl.ds(g * LANES, LANES)]
        for j in range(LANES):
          wvec = jnp.full((LANES,), wg[j], dtype=jnp.float32)
          e = g * LANES + j
          for s in range(D // LANES):
            sl = pl.ds(s * LANES, LANES)
            rbuf[e, sl] = rbuf[e, sl] * wvec
        return gcarry

      lax.fori_loop(0, EB // LANES, grp16, 0)

    npair = CH // 2

    def pairbody(p, carry):
      c0 = 2 * p
      c1 = c0 + 1
      g = c0 // GC
      q = g % 2
      j0 = c0 - g * GC

      pltpu.make_async_copy(
          xw_hbm.at[sidx.at[q].at[j0]], rows0, gsem0).wait()

      @pl.when(jnp.logical_and(j0 == 0, g + 1 < NG))
      def _():
        pltpu.async_copy(src_hbm.at[wid].at[g + 1], sidx.at[1 - q], isem)
        pltpu.async_copy(dst_hbm.at[wid].at[g + 1], didx.at[1 - q], isem)

      @pl.when(p > 0)
      def _():
        pltpu.make_async_copy(rows1, acc.at[didx.at[q].at[j0]],
                              ssem1).wait()

      pltpu.async_copy(xw_hbm.at[sidx.at[q].at[j0 + 1]], rows1, gsem1)
      scale(rows0, c0)
      pltpu.async_copy(rows0, acc.at[didx.at[q].at[j0]], ssem0, add=True)

      pltpu.make_async_copy(
          xw_hbm.at[sidx.at[q].at[j0 + 1]], rows1, gsem1).wait()

      @pl.when(p + 1 < npair)
      def _():
        nxt = c0 + 2
        gn = nxt // GC
        qn = gn % 2
        jn = nxt - gn * GC

        @pl.when(jn == 0)
        def _():
          pltpu.make_async_copy(src_hbm.at[wid].at[gn], sidx.at[qn],
                                isem).wait()
          pltpu.make_async_copy(dst_hbm.at[wid].at[gn], didx.at[qn],
                                isem).wait()

        pltpu.make_async_copy(rows0, acc.at[didx.at[q].at[j0]],
                              ssem0).wait()
        pltpu.async_copy(xw_hbm.at[sidx.at[qn].at[jn]], rows0, gsem0)

      scale(rows1, c1)
      pltpu.async_copy(rows1, acc.at[didx.at[q].at[j0 + 1]], ssem1,
                       add=True)
      return carry

    lax.fori_loop(0, npair, pairbody, 0)
    lg = (CH - 2) // GC
    pltpu.make_async_copy(
        rows0, acc.at[didx.at[lg % 2].at[CH - 2 - lg * GC]], ssem0).wait()
    pltpu.make_async_copy(
        rows1, acc.at[didx.at[lg % 2].at[CH - 1 - lg * GC]], ssem1).wait()
    plsc.subcore_barrier()
    sl = pl.ds(sid * rpt, rpt)
    pltpu.sync_copy(acc.at[sl], out_hbm.at[cid].at[sl])

  return agg_kernel


def _dinv_block(degp_blk):
  deg = jnp.sum(degp_blk, axis=1) + 1.0
  pos = deg > 0
  dinv = jnp.where(pos, lax.rsqrt(jnp.where(pos, deg, 1.0)), 0.0)
  return dinv[:, None]


def _xwp1_call(N, D, H, degp_t, x, W1):
  nb = N // BR

  def body(degp_ref, x_ref, w_ref, out_ref):
    dinv = _dinv_block(degp_ref[...])
    xw = lax.dot_general(x_ref[...], w_ref[...], (((1,), (1,)), ((), ())),
                         preferred_element_type=jnp.float32)
    out_ref[...] = xw * dinv

  return pl.pallas_call(
      body,
      grid=(nb,),
      in_specs=[
          pl.BlockSpec((BR, NC), lambda i: (i, 0)),
          pl.BlockSpec((BR, D), lambda i: (i, 0)),
          pl.BlockSpec((H, D), lambda i: (0, 0)),
      ],
      out_specs=pl.BlockSpec((BR, H), lambda i: (i, 0)),
      out_shape=jax.ShapeDtypeStruct((N, H), jnp.float32),
  )(degp_t, x, W1)


def _layer2_call(N, H, degp_t, aggp, xwp1, b1, W2):
  nb = N // BR

  def body(degp_ref, aggp_ref, xwp_ref, b_ref, w_ref, out_ref):
    dinv = _dinv_block(degp_ref[...])
    agg = aggp_ref[0] + aggp_ref[1] + xwp_ref[...]
    h = jnp.maximum(agg * dinv + b_ref[...], 0.0)
    xw2 = lax.dot_general(h, w_ref[...], (((1,), (1,)), ((), ())),
                          preferred_element_type=jnp.float32)
    out_ref[...] = xw2 * dinv

  return pl.pallas_call(
      body,
      grid=(nb,),
      in_specs=[
          pl.BlockSpec((BR, NC), lambda i: (i, 0)),
          pl.BlockSpec((NC, BR, H), lambda i: (0, i, 0)),
          pl.BlockSpec((BR, H), lambda i: (i, 0)),
          pl.BlockSpec((1, H), lambda i: (0, 0)),
          pl.BlockSpec((H, H), lambda i: (0, 0)),
      ],
      out_specs=pl.BlockSpec((BR, H), lambda i: (i, 0)),
      out_shape=jax.ShapeDtypeStruct((N, H), jnp.float32),
  )(degp_t, aggp, xwp1, b1, W2)


def _head_call(N, H, G, OUT, degp_t, aggp, xwp2, b2, batch2d,
               L1W, L1b, L2W, L2b):
  nb = N // BR

  def body(degp_ref, aggp_ref, xwp_ref, b_ref, batch_ref,
           l1w_ref, l1b_ref, l2w_ref, l2b_ref, out_ref, seg, cnt):
    i = pl.program_id(0)

    @pl.when(i == 0)
    def _():
      seg[...] = jnp.zeros_like(seg)
      cnt[...] = jnp.zeros_like(cnt)

    dinv = _dinv_block(degp_ref[...])
    agg = aggp_ref[0] + aggp_ref[1] + xwp_ref[...]
    h = jnp.maximum(agg * dinv + b_ref[...], 0.0)
    gids = lax.broadcasted_iota(jnp.int32, (BR, G), 1)
    oh = (batch_ref[...] == gids).astype(jnp.float32)
    seg[...] += lax.dot_general(oh, h, (((0,), (0,)), ((), ())),
                                preferred_element_type=jnp.float32)
    cnt[...] += jnp.sum(oh, axis=0)[:, None]

    @pl.when(i == nb - 1)
    def _():
      g = seg[...] / jnp.clip(cnt[...], 1.0)
      z = lax.dot_general(g, l1w_ref[...], (((1,), (1,)), ((), ())),
                          preferred_element_type=jnp.float32)
      z = jnp.maximum(z + l1b_ref[...], 0.0)
      o = lax.dot_general(z, l2w_ref[...], (((1,), (1,)), ((), ())),
                          preferred_element_type=jnp.float32)
      out_ref[...] = o + l2b_ref[...]

  return pl.pallas_call(
      body,
      grid=(nb,),
      in_specs=[
          pl.BlockSpec((BR, NC), lambda i: (i, 0)),
          pl.BlockSpec((NC, BR, H), lambda i: (0, i, 0)),
          pl.BlockSpec((BR, H), lambda i: (i, 0)),
          pl.BlockSpec((1, H), lambda i: (0, 0)),
          pl.BlockSpec((BR, 1), lambda i: (i, 0)),
          pl.BlockSpec((H, H), lambda i: (0, 0)),
          pl.BlockSpec((1, H), lambda i: (0, 0)),
          pl.BlockSpec((OUT, H), lambda i: (0, 0)),
          pl.BlockSpec((1, OUT), lambda i: (0, 0)),
      ],
      out_specs=pl.BlockSpec((G, OUT), lambda i: (0, 0)),
      out_shape=jax.ShapeDtypeStruct((G, OUT), jnp.float32),
      scratch_shapes=[
          pltpu.VMEM((G, H), jnp.float32),
          pltpu.VMEM((G, H), jnp.float32),
      ],
  )(degp_t, aggp, xwp2, b2, batch2d, L1W, L1b, L2W, L2b)


def kernel(x, edge_index, edge_weight, batch, W1, b1, W2, b2,
           L1W, L1b, L2W, L2b):
  N, D = x.shape
  H = W1.shape[0]
  OUT = L2W.shape[0]
  G = 16
  E = edge_index.shape[1]

  src = edge_index[0].astype(jnp.int32)
  dst = edge_index[1].astype(jnp.int32)
  w = edge_weight.astype(jnp.float32)
  ntiles = NC * NS
  ch = -(-E // (ntiles * EB))
  ch = -(-ch // (2 * GC)) * (2 * GC)
  ng = ch // GC
  ep = ntiles * ch * EB
  pad = ep - E
  if pad:
    spread = jnp.arange(pad, dtype=jnp.int32) % jnp.int32(N)
    src = jnp.concatenate([src, spread])
    dst = jnp.concatenate([dst, spread])
    w = jnp.concatenate([w, jnp.zeros((pad,), jnp.float32)])
  src_r = src.reshape(ntiles, ng, GC, EB)
  dst_r = dst.reshape(ntiles, ng, GC, EB)
  dst_r2 = dst.reshape(ntiles, ch, EB)
  w_r = w.reshape(ntiles, ch, EB)
  npad = -(-N // (NS * 8)) * NS * 8
  zeros_n = jnp.zeros((N,), jnp.float32)
  zrows = jnp.zeros((npad // NS, D), jnp.float32)
  batch2d = batch.astype(jnp.int32).reshape(N, 1)
  b1r = b1.reshape(1, H)
  b2r = b2.reshape(1, H)
  l1br = L1b.reshape(1, D)
  l2br = L2b.reshape(1, OUT)

  degp = _make_deg_kernel(N, ch)(dst_r2, w_r, zeros_n)
  degp_t = degp.T

  xwp1 = _xwp1_call(N, D, H, degp_t, x, W1)
  aggp1 = _make_agg_kernel(npad, ch, H)(xwp1, src_r, dst_r, w_r, zrows)

  xwp2 = _layer2_call(N, H, degp_t, aggp1[:, :N], xwp1, b1r, W2)
  aggp2 = _make_agg_kernel(npad, ch, H)(xwp2, src_r, dst_r, w_r, zrows)

  return _head_call(N, H, G, OUT, degp_t, aggp2[:, :N], xwp2, b2r, batch2d,
                    L1W, l1br, L2W, l2br)


# speedup vs baseline: 1.0037x; 1.0037x over previous
import functools

import jax
import jax.numpy as jnp
from jax import lax
from jax.experimental import pallas as pl
from jax.experimental.pallas import tpu as pltpu
from jax.experimental.pallas import tpu_sc as plsc

NC = 2
NS = 16
LANES = 16
EB = 128
GC = 8
BR = 1000


def _sc_mesh():
  return plsc.VectorSubcoreMesh(
      core_axis_name="c", subcore_axis_name="s", num_cores=NC,
      num_subcores=NS)


def _make_deg_kernel(N, CH):
  @functools.partial(
      pl.kernel,
      out_type=jax.ShapeDtypeStruct((NC, N), jnp.float32),
      mesh=_sc_mesh(),
      scratch_types=[
          pltpu.VMEM((CH, EB), jnp.int32),
          pltpu.VMEM((CH, EB), jnp.float32),
          pltpu.VMEM_SHARED((N,), jnp.float32),
      ],
  )
  def deg_kernel(dst_hbm, w_hbm, zeros_hbm, out_hbm, dstv, wv, deg_sh):
    cid = lax.axis_index("c")
    sid = lax.axis_index("s")
    wid = sid * NC + cid
    pltpu.sync_copy(dst_hbm.at[wid], dstv)
    pltpu.sync_copy(w_hbm.at[wid], wv)

    @pl.when(sid == 0)
    def _():
      pltpu.sync_copy(zeros_hbm, deg_sh)

    plsc.subcore_barrier()

    def chunk(c, carry):
      pltpu.sync_copy(wv.at[c], deg_sh.at[dstv.at[c]], add=True)
      return carry

    lax.fori_loop(0, CH, chunk, 0)
    plsc.subcore_barrier()

    @pl.when(sid == 0)
    def _():
      pltpu.sync_copy(deg_sh, out_hbm.at[cid])

  return deg_kernel


def _make_agg_kernel(NP, CH, D):
  rpt = NP // NS

  assert CH % (2 * GC) == 0
  NG = CH // GC

  @functools.partial(
      pl.kernel,
      out_type=jax.ShapeDtypeStruct((NC, NP, D), jnp.float32),
      mesh=_sc_mesh(),
      scratch_types=[
          pltpu.VMEM((2, GC, EB), jnp.int32),
          pltpu.VMEM((2, GC, EB), jnp.int32),
          pltpu.VMEM((CH, EB), jnp.float32),
          pltpu.VMEM((EB, D), jnp.float32),
          pltpu.VMEM((EB, D), jnp.float32),
          pltpu.VMEM_SHARED((NP, D), jnp.float32),
          pltpu.SemaphoreType.DMA,
          pltpu.SemaphoreType.DMA,
          pltpu.SemaphoreType.DMA,
      ],
  )
  def agg_kernel(xw_hbm, src_hbm, dst_hbm, w_hbm, zrows_hbm, out_hbm,
                 sidx, didx, wv, rows0, rows1, acc,
                 gsem0, gsem1, isem):
    cid = lax.axis_index("c")
    sid = lax.axis_index("s")
    wid = sid * NC + cid

    pltpu.sync_copy(w_hbm.at[wid], wv)
    pltpu.sync_copy(src_hbm.at[wid].at[0], sidx.at[0])
    pltpu.sync_copy(dst_hbm.at[wid].at[0], didx.at[0])
    pltpu.async_copy(xw_hbm.at[sidx.at[0].at[0]], rows0, gsem0)
    pltpu.sync_copy(zrows_hbm, acc.at[pl.ds(sid * rpt, rpt)])
    plsc.subcore_barrier()

    def scale(rbuf, c):
      def grp16(g):
        wg = wv[c, pl.ds(g * LANES, LANES)]
        for j in range(LANES):
          wvec = jnp.full((LANES,), wg[j], dtype=jnp.float32)
          e = g * LANES + j
          for s in range(D // LANES):
            sl = pl.ds(s * LANES, LANES)
            rbuf[e, sl] = rbuf[e, sl] * wvec

      plsc.parallel_loop(0, EB // LANES, 1, unroll=2)(grp16)

    npair = CH // 2

    def pairbody(p, carry):
      c0 = 2 * p
      c1 = c0 + 1
      g = c0 // GC
      q = g % 2
      j0 = c0 - g * GC

      pltpu.make_async_copy(
          xw_hbm.at[sidx.at[q].at[j0]], rows0, gsem0).wait()

      @pl.when(jnp.logical_and(j0 == 0, g + 1 < NG))
      def _():
        pltpu.async_copy(src_hbm.at[wid].at[g + 1], sidx.at[1 - q], isem)
        pltpu.async_copy(dst_hbm.at[wid].at[g + 1], didx.at[1 - q], isem)

      pltpu.async_copy(xw_hbm.at[sidx.at[q].at[j0 + 1]], rows1, gsem1)
      scale(rows0, c0)
      pltpu.sync_copy(rows0, acc.at[didx.at[q].at[j0]], add=True)

      pltpu.make_async_copy(
          xw_hbm.at[sidx.at[q].at[j0 + 1]], rows1, gsem1).wait()

      @pl.when(p + 1 < npair)
      def _():
        nxt = c0 + 2
        gn = nxt // GC
        qn = gn % 2
        jn = nxt - gn * GC

        @pl.when(jn == 0)
        def _():
          pltpu.make_async_copy(src_hbm.at[wid].at[gn], sidx.at[qn],
                                isem).wait()
          pltpu.make_async_copy(dst_hbm.at[wid].at[gn], didx.at[qn],
                                isem).wait()

        pltpu.async_copy(xw_hbm.at[sidx.at[qn].at[jn]], rows0, gsem0)

      scale(rows1, c1)
      pltpu.sync_copy(rows1, acc.at[didx.at[q].at[j0 + 1]], add=True)
      return carry

    lax.fori_loop(0, npair, pairbody, 0)
    plsc.subcore_barrier()
    sl = pl.ds(sid * rpt, rpt)
    pltpu.sync_copy(acc.at[sl], out_hbm.at[cid].at[sl])

  return agg_kernel


def _dinv_block(degp_blk):
  deg = jnp.sum(degp_blk, axis=1) + 1.0
  pos = deg > 0
  dinv = jnp.where(pos, lax.rsqrt(jnp.where(pos, deg, 1.0)), 0.0)
  return dinv[:, None]


def _xwp1_call(N, D, H, degp_t, x, W1):
  nb = N // BR

  def body(degp_ref, x_ref, w_ref, out_ref):
    dinv = _dinv_block(degp_ref[...])
    xw = lax.dot_general(x_ref[...], w_ref[...], (((1,), (1,)), ((), ())),
                         preferred_element_type=jnp.float32)
    out_ref[...] = xw * dinv

  return pl.pallas_call(
      body,
      grid=(nb,),
      in_specs=[
          pl.BlockSpec((BR, NC), lambda i: (i, 0)),
          pl.BlockSpec((BR, D), lambda i: (i, 0)),
          pl.BlockSpec((H, D), lambda i: (0, 0)),
      ],
      out_specs=pl.BlockSpec((BR, H), lambda i: (i, 0)),
      out_shape=jax.ShapeDtypeStruct((N, H), jnp.float32),
  )(degp_t, x, W1)


def _layer2_call(N, H, degp_t, aggp, xwp1, b1, W2):
  nb = N // BR

  def body(degp_ref, aggp_ref, xwp_ref, b_ref, w_ref, out_ref):
    dinv = _dinv_block(degp_ref[...])
    agg = aggp_ref[0] + aggp_ref[1] + xwp_ref[...]
    h = jnp.maximum(agg * dinv + b_ref[...], 0.0)
    xw2 = lax.dot_general(h, w_ref[...], (((1,), (1,)), ((), ())),
                          preferred_element_type=jnp.float32)
    out_ref[...] = xw2 * dinv

  return pl.pallas_call(
      body,
      grid=(nb,),
      in_specs=[
          pl.BlockSpec((BR, NC), lambda i: (i, 0)),
          pl.BlockSpec((NC, BR, H), lambda i: (0, i, 0)),
          pl.BlockSpec((BR, H), lambda i: (i, 0)),
          pl.BlockSpec((1, H), lambda i: (0, 0)),
          pl.BlockSpec((H, H), lambda i: (0, 0)),
      ],
      out_specs=pl.BlockSpec((BR, H), lambda i: (i, 0)),
      out_shape=jax.ShapeDtypeStruct((N, H), jnp.float32),
  )(degp_t, aggp, xwp1, b1, W2)


def _head_call(N, H, G, OUT, degp_t, aggp, xwp2, b2, batch2d,
               L1W, L1b, L2W, L2b):
  nb = N // BR

  def body(degp_ref, aggp_ref, xwp_ref, b_ref, batch_ref,
           l1w_ref, l1b_ref, l2w_ref, l2b_ref, out_ref, seg, cnt):
    i = pl.program_id(0)

    @pl.when(i == 0)
    def _():
      seg[...] = jnp.zeros_like(seg)
      cnt[...] = jnp.zeros_like(cnt)

    dinv = _dinv_block(degp_ref[...])
    agg = aggp_ref[0] + aggp_ref[1] + xwp_ref[...]
    h = jnp.maximum(agg * dinv + b_ref[...], 0.0)
    gids = lax.broadcasted_iota(jnp.int32, (BR, G), 1)
    oh = (batch_ref[...] == gids).astype(jnp.float32)
    seg[...] += lax.dot_general(oh, h, (((0,), (0,)), ((), ())),
                                preferred_element_type=jnp.float32)
    cnt[...] += jnp.sum(oh, axis=0)[:, None]

    @pl.when(i == nb - 1)
    def _():
      g = seg[...] / jnp.clip(cnt[...], 1.0)
      z = lax.dot_general(g, l1w_ref[...], (((1,), (1,)), ((), ())),
                          preferred_element_type=jnp.float32)
      z = jnp.maximum(z + l1b_ref[...], 0.0)
      o = lax.dot_general(z, l2w_ref[...], (((1,), (1,)), ((), ())),
                          preferred_element_type=jnp.float32)
      out_ref[...] = o + l2b_ref[...]

  return pl.pallas_call(
      body,
      grid=(nb,),
      in_specs=[
          pl.BlockSpec((BR, NC), lambda i: (i, 0)),
          pl.BlockSpec((NC, BR, H), lambda i: (0, i, 0)),
          pl.BlockSpec((BR, H), lambda i: (i, 0)),
          pl.BlockSpec((1, H), lambda i: (0, 0)),
          pl.BlockSpec((BR, 1), lambda i: (i, 0)),
          pl.BlockSpec((H, H), lambda i: (0, 0)),
          pl.BlockSpec((1, H), lambda i: (0, 0)),
          pl.BlockSpec((OUT, H), lambda i: (0, 0)),
          pl.BlockSpec((1, OUT), lambda i: (0, 0)),
      ],
      out_specs=pl.BlockSpec((G, OUT), lambda i: (0, 0)),
      out_shape=jax.ShapeDtypeStruct((G, OUT), jnp.float32),
      scratch_shapes=[
          pltpu.VMEM((G, H), jnp.float32),
          pltpu.VMEM((G, H), jnp.float32),
      ],
  )(degp_t, aggp, xwp2, b2, batch2d, L1W, L1b, L2W, L2b)


def kernel(x, edge_index, edge_weight, batch, W1, b1, W2, b2,
           L1W, L1b, L2W, L2b):
  N, D = x.shape
  H = W1.shape[0]
  OUT = L2W.shape[0]
  G = 16
  E = edge_index.shape[1]

  src = edge_index[0].astype(jnp.int32)
  dst = edge_index[1].astype(jnp.int32)
  w = edge_weight.astype(jnp.float32)
  ntiles = NC * NS
  ch = -(-E // (ntiles * EB))
  ch = -(-ch // (2 * GC)) * (2 * GC)
  ng = ch // GC
  ep = ntiles * ch * EB
  pad = ep - E
  if pad:
    spread = jnp.arange(pad, dtype=jnp.int32) % jnp.int32(N)
    src = jnp.concatenate([src, spread])
    dst = jnp.concatenate([dst, spread])
    w = jnp.concatenate([w, jnp.zeros((pad,), jnp.float32)])
  src_r = src.reshape(ntiles, ng, GC, EB)
  dst_r = dst.reshape(ntiles, ng, GC, EB)
  dst_r2 = dst.reshape(ntiles, ch, EB)
  w_r = w.reshape(ntiles, ch, EB)
  npad = -(-N // (NS * 8)) * NS * 8
  zeros_n = jnp.zeros((N,), jnp.float32)
  zrows = jnp.zeros((npad // NS, D), jnp.float32)
  batch2d = batch.astype(jnp.int32).reshape(N, 1)
  b1r = b1.reshape(1, H)
  b2r = b2.reshape(1, H)
  l1br = L1b.reshape(1, D)
  l2br = L2b.reshape(1, OUT)

  degp = _make_deg_kernel(N, ch)(dst_r2, w_r, zeros_n)
  degp_t = degp.T

  xwp1 = _xwp1_call(N, D, H, degp_t, x, W1)
  aggp1 = _make_agg_kernel(npad, ch, H)(xwp1, src_r, dst_r, w_r, zrows)

  xwp2 = _layer2_call(N, H, degp_t, aggp1[:, :N], xwp1, b1r, W2)
  aggp2 = _make_agg_kernel(npad, ch, H)(xwp2, src_r, dst_r, w_r, zrows)

  return _head_call(N, H, G, OUT, degp_t, aggp2[:, :N], xwp2, b2r, batch2d,
                    L1W, l1br, L2W, l2br)
